# Initial kernel scaffold; baseline (speedup 1.0000x reference)
import functools

import jax
import jax.numpy as jnp
from jax import lax
from jax.experimental import pallas as pl
from jax.experimental.pallas import tpu as pltpu

_NODES = 100000
_G = 64
_D = 128
_BLK = 10000
_NBLK = _NODES // _BLK
_LOG2 = 0.6931471805599453


def _ep_sum(t):
    a = -t
    sp = jnp.maximum(a, 0.0) + jnp.log1p(jnp.exp(-jnp.abs(a)))
    return jnp.sum(_LOG2 - sp)


def _body(b1_ref, b2_ref, z1_ref, z2_ref, out_ref, g1_ref, g2_ref):
    phase = pl.program_id(0)
    i = pl.program_id(1)

    b1 = b1_ref[0]
    b2 = b2_ref[0]

    @pl.when((phase == 0) & (i == 0))
    def _init():
        g1_ref[...] = jnp.zeros_like(g1_ref)
        g2_ref[...] = jnp.zeros_like(g2_ref)
        out_ref[...] = jnp.zeros_like(out_ref)

    @pl.when(phase == 0)
    def _accumulate_g():
        iota_g = lax.broadcasted_iota(jnp.int32, (_G, _BLK), 0)
        oh1t = (iota_g == jnp.broadcast_to(b1, (_G, _BLK))).astype(jnp.float32)
        oh2t = (iota_g == jnp.broadcast_to(b2, (_G, _BLK))).astype(jnp.float32)
        g1_ref[...] += lax.dot(oh1t, z1_ref[...],
                               precision=lax.Precision.HIGHEST)
        g2_ref[...] += lax.dot(oh2t, z2_ref[...],
                               precision=lax.Precision.HIGHEST)

    @pl.when(phase == 1)
    def _similarity():
        gcat = jnp.concatenate([g1_ref[...], g2_ref[...]], axis=0)
        s1 = lax.dot_general(z1_ref[...], gcat, (((1,), (1,)), ((), ())),
                             precision=lax.Precision.DEFAULT)
        s2 = lax.dot_general(z2_ref[...], gcat, (((1,), (1,)), ((), ())),
                             precision=lax.Precision.DEFAULT)
        iota_n = lax.broadcasted_iota(jnp.int32, (_BLK, _G), 1)
        oh1 = (iota_n == jnp.broadcast_to(b1.T, (_BLK, _G))).astype(jnp.float32)
        oh2 = (iota_n == jnp.broadcast_to(b2.T, (_BLK, _G))).astype(jnp.float32)
        t11 = jnp.sum(s1[:, :_G] * oh1, axis=1)
        t12 = jnp.sum(s1[:, _G:] * oh1, axis=1)
        t21 = jnp.sum(s2[:, :_G] * oh2, axis=1)
        t22 = jnp.sum(s2[:, _G:] * oh2, axis=1)
        block = (_ep_sum(t11) - _ep_sum(t12)) - (_ep_sum(t22) - _ep_sum(t21))
        out_ref[0, 0] += block


@jax.jit
def kernel(batch_1, batch_2, z1, z2):
    b1r = batch_1.astype(jnp.int32).reshape(_NBLK, 1, _BLK)
    b2r = batch_2.astype(jnp.int32).reshape(_NBLK, 1, _BLK)
    out = pl.pallas_call(
        _body,
        grid=(2, _NBLK),
        in_specs=[
            pl.BlockSpec((1, 1, _BLK), lambda p, i: (i, 0, 0)),
            pl.BlockSpec((1, 1, _BLK), lambda p, i: (i, 0, 0)),
            pl.BlockSpec((_BLK, _D), lambda p, i: (i, 0)),
            pl.BlockSpec((_BLK, _D), lambda p, i: (i, 0)),
        ],
        out_specs=pl.BlockSpec((1, 1), lambda p, i: (0, 0)),
        out_shape=jax.ShapeDtypeStruct((1, 1), jnp.float32),
        scratch_shapes=[
            pltpu.VMEM((_G, _D), jnp.float32),
            pltpu.VMEM((_G, _D), jnp.float32),
        ],
        compiler_params=pltpu.CompilerParams(
            dimension_semantics=("arbitrary", "arbitrary"),
        ),
    )(b1r, b2r, z1, z2)
    return out[0, 0]


# speedup vs baseline: 2.4208x; 2.4208x over previous
import functools

import jax
import jax.numpy as jnp
from jax import lax
from jax.experimental import pallas as pl
from jax.experimental.pallas import tpu as pltpu

_NODES = 100000
_G = 64
_D = 128
_BLK = 10000
_NBLK = _NODES // _BLK
_LOG2 = 0.6931471805599453


def _ep_sum(t):
    a = -t
    sp = jnp.maximum(a, 0.0) + jnp.log1p(jnp.exp(-jnp.abs(a)))
    return jnp.sum(_LOG2 - sp)


def _body(b1_ref, b2_ref, z1_ref, z2_ref, out_ref, g1_ref, g2_ref,
          acc_ref, comp_ref):
    phase = pl.program_id(0)
    i = pl.program_id(1)

    b1 = b1_ref[0]
    b2 = b2_ref[0]

    @pl.when((phase == 0) & (i == 0))
    def _init():
        g1_ref[...] = jnp.zeros_like(g1_ref)
        g2_ref[...] = jnp.zeros_like(g2_ref)
        out_ref[...] = jnp.zeros_like(out_ref)
        acc_ref[...] = jnp.zeros_like(acc_ref)
        comp_ref[...] = jnp.zeros_like(comp_ref)

    @pl.when(phase == 0)
    def _accumulate_g():
        iota_g = lax.broadcasted_iota(jnp.int32, (_G, _BLK), 0)
        oh1t = (iota_g == jnp.broadcast_to(b1, (_G, _BLK))).astype(jnp.float32)
        oh2t = (iota_g == jnp.broadcast_to(b2, (_G, _BLK))).astype(jnp.float32)
        g1_ref[...] += lax.dot(oh1t, z1_ref[...],
                               precision=lax.Precision.HIGHEST)
        g2_ref[...] += lax.dot(oh2t, z2_ref[...],
                               precision=lax.Precision.HIGHEST)

    @pl.when(phase == 1)
    def _similarity():
        gcat = jnp.concatenate([g1_ref[...], g2_ref[...]],
                               axis=0).astype(jnp.bfloat16)
        z1b = z1_ref[...].astype(jnp.bfloat16)
        z2b = z2_ref[...].astype(jnp.bfloat16)
        s1 = lax.dot_general(z1b, gcat, (((1,), (1,)), ((), ())),
                             preferred_element_type=jnp.float32)
        s2 = lax.dot_general(z2b, gcat, (((1,), (1,)), ((), ())),
                             preferred_element_type=jnp.float32)
        iota_n = lax.broadcasted_iota(jnp.int32, (_BLK, _G), 1)
        oh1 = (iota_n == jnp.broadcast_to(b1.T, (_BLK, _G))).astype(jnp.float32)
        oh2 = (iota_n == jnp.broadcast_to(b2.T, (_BLK, _G))).astype(jnp.float32)
        t11 = jnp.sum(s1[:, :_G] * oh1, axis=1)
        t12 = jnp.sum(s1[:, _G:] * oh1, axis=1)
        t21 = jnp.sum(s2[:, :_G] * oh2, axis=1)
        t22 = jnp.sum(s2[:, _G:] * oh2, axis=1)
        blk = jnp.concatenate([
            jnp.reshape(_ep_sum(t11), (1, 1)),
            jnp.reshape(_ep_sum(t12), (1, 1)),
            jnp.reshape(_ep_sum(t22), (1, 1)),
            jnp.reshape(_ep_sum(t21), (1, 1)),
        ], axis=1)
        y = blk - comp_ref[...]
        t = acc_ref[...] + y
        comp_ref[...] = (t - acc_ref[...]) - y
        acc_ref[...] = t

        @pl.when(i == _NBLK - 1)
        def _finish():
            total = acc_ref[...] - comp_ref[...]
            a11 = total[0:1, 0:1]
            a12 = total[0:1, 1:2]
            a22 = total[0:1, 2:3]
            a21 = total[0:1, 3:4]
            out_ref[...] = (a11 - a12) - (a22 - a21)


@jax.jit
def kernel(batch_1, batch_2, z1, z2):
    b1r = batch_1.astype(jnp.int32).reshape(_NBLK, 1, _BLK)
    b2r = batch_2.astype(jnp.int32).reshape(_NBLK, 1, _BLK)
    out = pl.pallas_call(
        _body,
        grid=(2, _NBLK),
        in_specs=[
            pl.BlockSpec((1, 1, _BLK), lambda p, i: (i, 0, 0)),
            pl.BlockSpec((1, 1, _BLK), lambda p, i: (i, 0, 0)),
            pl.BlockSpec((_BLK, _D), lambda p, i: (i, 0)),
            pl.BlockSpec((_BLK, _D), lambda p, i: (i, 0)),
        ],
        out_specs=pl.BlockSpec((1, 1), lambda p, i: (0, 0)),
        out_shape=jax.ShapeDtypeStruct((1, 1), jnp.float32),
        scratch_shapes=[
            pltpu.VMEM((_G, _D), jnp.float32),
            pltpu.VMEM((_G, _D), jnp.float32),
            pltpu.VMEM((1, 4), jnp.float32),
            pltpu.VMEM((1, 4), jnp.float32),
        ],
        compiler_params=pltpu.CompilerParams(
            dimension_semantics=("arbitrary", "arbitrary"),
        ),
    )(b1r, b2r, z1, z2)
    return out[0, 0]
